# baseline (speedup 1.0000x reference)
import functools

import jax
import jax.numpy as jnp
from jax import lax
from jax.experimental import pallas as pl
from jax.experimental.pallas import tpu as pltpu
from jax.experimental.pallas import tpu_sc as plsc

_VOCAB = 1000000
_DIM = 64
_PADW = 128
_LANES = 16
_NC = 2
_NS = 16
_NW = _NC * _NS
_GRP = 128
_NBUF = 4
_NACC = 8
_RU = 4

_VCH = 128
_NCH = _VOCAB // _VCH
_VREM = _VOCAB - _NCH * _VCH
_CPW = _NCH // _NW
_CEX = _NCH - _CPW * _NW
_NPAIR = (_CPW + 2) // 2

_mesh = plsc.VectorSubcoreMesh(core_axis_name="c", subcore_axis_name="s")


@functools.partial(
    pl.kernel,
    mesh=_mesh,
    out_type=(jax.ShapeDtypeStruct((_VOCAB, _PADW), jnp.float32),
              jax.ShapeDtypeStruct((_VOCAB, _PADW), jnp.float32)),
    scratch_types=[
        pltpu.VMEM((_VCH, _DIM), jnp.float32),
        pltpu.VMEM((_VCH, _DIM), jnp.float32),
        pltpu.VMEM((_VCH, _PADW), jnp.float32),
        pltpu.VMEM((_VCH, _PADW), jnp.float32),
        pltpu.SemaphoreType.DMA,
        pltpu.SemaphoreType.DMA,
        pltpu.SemaphoreType.DMA,
        pltpu.SemaphoreType.DMA,
    ],
)
def _detile_tables(u_t, v_t, up, vp, vin0, vin1, vout0, vout1,
                   semi0, semi1, semo0, semo1):
    wid = lax.axis_index("s") * _NC + lax.axis_index("c")
    start = wid * _CPW + jnp.minimum(wid, _CEX)
    vins = (vin0, vin1)
    vouts = (vout0, vout1)
    semis = (semi0, semi1)
    semos = (semo0, semo1)

    def chunk(t):
        return jnp.minimum(start + t, _NCH - 1)

    def issue_in(src, c, b):
        pltpu.async_copy(src.at[pl.ds(c * _VCH, _VCH)], vins[b], semis[b])

    def drain_in(b):
        pltpu.make_async_copy(u_t.at[pl.ds(0, _VCH)], vins[b],
                              semis[b]).wait()

    def issue_out(dst, c, b):
        pltpu.async_copy(vouts[b], dst.at[pl.ds(c * _VCH, _VCH)], semos[b])

    def drain_out(b):
        pltpu.make_async_copy(up.at[pl.ds(0, _VCH)], vouts[b],
                              semos[b]).wait()

    def shuttle(b, rows):
        def body(i, carry):
            for r in range(_RU):
                row = i * _RU + r
                for s in range(_DIM // _LANES):
                    sl = pl.ds(s * _LANES, _LANES)
                    vouts[b][row, sl] = vins[b][row, sl]
            return carry

        lax.fori_loop(0, rows // _RU, body, 0)

    def do_table(src, dst):
        issue_in(src, chunk(0), 0)

        def pair_body(p, carry):
            issue_in(src, chunk(2 * p + 1), 1)
            drain_in(0)

            @pl.when(p > 0)
            def _():
                drain_out(0)

            shuttle(0, _VCH)
            issue_out(dst, chunk(2 * p), 0)
            issue_in(src, chunk(2 * p + 2), 0)
            drain_in(1)

            @pl.when(p > 0)
            def _():
                drain_out(1)

            shuttle(1, _VCH)
            issue_out(dst, chunk(2 * p + 1), 1)
            return carry

        lax.fori_loop(0, _NPAIR, pair_body, 0)
        drain_in(0)
        drain_out(0)
        drain_out(1)

    do_table(u_t, up)
    do_table(v_t, vp)

    @pl.when(wid == 0)
    def _():
        for src, dst in ((u_t, up), (v_t, vp)):
            pltpu.sync_copy(src.at[pl.ds(_NCH * _VCH, _VREM)],
                            vin0.at[pl.ds(0, _VREM)])
            shuttle(0, _VREM)
            pltpu.sync_copy(vout0.at[pl.ds(0, _VREM)],
                            dst.at[pl.ds(_NCH * _VCH, _VREM)])


def _make_bias_kernel(n_groups_total):
    n_groups_w = n_groups_total // _NW

    @functools.partial(
        pl.kernel,
        mesh=_mesh,
        out_type=jax.ShapeDtypeStruct((_NW, _LANES), jnp.float32),
        compiler_params=pltpu.CompilerParams(use_tc_tiling_on_sc=False),
        scratch_types=[
            pltpu.VMEM((n_groups_w, _GRP), jnp.int32),
            pltpu.VMEM((n_groups_w, _GRP), jnp.int32),
            pltpu.VMEM((_NBUF, _GRP), jnp.float32),
            pltpu.VMEM((_NBUF, _GRP), jnp.float32),
            pltpu.VMEM((_LANES,), jnp.float32),
        ] + [pltpu.SemaphoreType.DMA] * _NBUF,
    )
    def bias_kernel(tok_hbm, head_hbm, ub_hbm, vb_hbm,
                    out_hbm, idx_t, idx_h, ub, vb, acc_v, *sems):
        wid = lax.axis_index("s") * _NC + lax.axis_index("c")
        g_base = wid * n_groups_w
        pltpu.sync_copy(tok_hbm.at[pl.ds(g_base, n_groups_w)], idx_t)
        pltpu.sync_copy(head_hbm.at[pl.ds(g_base, n_groups_w)], idx_h)

        def issue(gi, b):
            pltpu.async_copy(ub_hbm.at[idx_t.at[gi]], ub.at[b], sems[b])
            pltpu.async_copy(vb_hbm.at[idx_h.at[gi]], vb.at[b], sems[b])

        def drain(b):
            pltpu.make_async_copy(ub_hbm.at[pl.ds(0, _GRP)],
                                  ub.at[b], sems[b]).wait()
            pltpu.make_async_copy(vb_hbm.at[pl.ds(0, _GRP)],
                                  vb.at[b], sems[b]).wait()

        def compute(b, accs):
            a = list(accs)
            for j in range(_GRP // _LANES):
                sl = pl.ds(j * _LANES, _LANES)
                a[j] = a[j] + ub[b, sl] + vb[b, sl]
            return tuple(a)

        for b in range(_NBUF - 1):
            issue(b, b)

        def quad_body(q, accs):
            for b in range(_NBUF):
                gi = q * _NBUF + b
                issue(jnp.minimum(gi + _NBUF - 1, n_groups_w - 1),
                      (b + _NBUF - 1) % _NBUF)
                drain(b)
                accs = compute(b, accs)
            return accs

        accs = lax.fori_loop(
            0, n_groups_w // _NBUF, quad_body,
            tuple(jnp.zeros((_LANES,), jnp.float32) for _ in range(_NACC)))
        for b in range(_NBUF - 1):
            drain(b)

        total = accs[0]
        for a in accs[1:]:
            total = total + a
        acc_v[...] = total
        pltpu.sync_copy(acc_v, out_hbm.at[wid])

    return bias_kernel


def _make_row_kernel(n_groups_total):
    n_groups_w = n_groups_total // _NW
    assert n_groups_w % 2 == 0

    @functools.partial(
        pl.kernel,
        mesh=_mesh,
        out_type=jax.ShapeDtypeStruct((_NW, _LANES), jnp.float32),
        scratch_types=[
            pltpu.VMEM((n_groups_w, _GRP), jnp.int32),
            pltpu.VMEM((n_groups_w, _GRP), jnp.int32),
            pltpu.VMEM((2, _GRP, _PADW), jnp.float32),
            pltpu.VMEM((2, _GRP, _PADW), jnp.float32),
            pltpu.VMEM((_LANES,), jnp.float32),
            pltpu.SemaphoreType.DMA,
            pltpu.SemaphoreType.DMA,
        ],
    )
    def row_kernel(tok_hbm, head_hbm, u_hbm, v_hbm,
                   out_hbm, idx_t, idx_h, u_rows, v_rows, acc_v,
                   sem0, sem1):
        wid = lax.axis_index("s") * _NC + lax.axis_index("c")
        g_base = wid * n_groups_w
        pltpu.sync_copy(tok_hbm.at[pl.ds(g_base, n_groups_w)], idx_t)
        pltpu.sync_copy(head_hbm.at[pl.ds(g_base, n_groups_w)], idx_h)
        sems = (sem0, sem1)

        def issue(gi, b):
            pltpu.async_copy(u_hbm.at[idx_t.at[gi]], u_rows.at[b], sems[b])
            pltpu.async_copy(v_hbm.at[idx_h.at[gi]], v_rows.at[b], sems[b])

        def drain(b):
            pltpu.make_async_copy(u_hbm.at[pl.ds(0, _GRP)],
                                  u_rows.at[b], sems[b]).wait()
            pltpu.make_async_copy(v_hbm.at[pl.ds(0, _GRP)],
                                  v_rows.at[b], sems[b]).wait()

        def compute(b, accs):
            def row_body(i, a):
                a = list(a)
                for r in range(_RU):
                    for s in range(_DIM // _LANES):
                        sl = pl.ds(s * _LANES, _LANES)
                        k = (r % 2) * (_DIM // _LANES) + s
                        a[k] = a[k] + (u_rows[b, i * _RU + r, sl] *
                                       v_rows[b, i * _RU + r, sl])
                return tuple(a)

            return lax.fori_loop(0, _GRP // _RU, row_body, accs)

        issue(0, 0)

        def pair_body(p, accs):
            issue(p * 2 + 1, 1)
            drain(0)
            accs = compute(0, accs)
            issue(jnp.minimum(p * 2 + 2, n_groups_w - 1), 0)
            drain(1)
            return compute(1, accs)

        accs = lax.fori_loop(
            0, n_groups_w // 2, pair_body,
            tuple(jnp.zeros((_LANES,), jnp.float32) for _ in range(_NACC)))
        drain(0)

        total = accs[0]
        for a in accs[1:]:
            total = total + a
        acc_v[...] = total
        pltpu.sync_copy(acc_v, out_hbm.at[wid])

    return row_kernel


def kernel(tokens_batch, heads_batch, U, Ubias, V, Vbias):
    b, l = tokens_batch.shape
    n = b * l
    n_groups_total = n // _GRP
    tok = tokens_batch.reshape(n_groups_total, _GRP).astype(jnp.int32)
    head = heads_batch.reshape(n_groups_total, _GRP).astype(jnp.int32)
    ub_flat = Ubias.reshape(-1)
    vb_flat = Vbias.reshape(-1)
    bias_partials = _make_bias_kernel(n_groups_total)(
        tok, head, ub_flat, vb_flat)
    u_wide, v_wide = _detile_tables(U, V)
    partials = _make_row_kernel(n_groups_total)(tok, head, u_wide, v_wide)
    return jnp.sum(partials) + jnp.sum(bias_partials)


# speedup vs baseline: 1.4773x; 1.4773x over previous
import functools

import jax
import jax.numpy as jnp
from jax import lax
from jax.experimental import pallas as pl
from jax.experimental.pallas import tpu as pltpu
from jax.experimental.pallas import tpu_sc as plsc

_VOCAB = 1000000
_DIM = 64
_LANES = 16
_NC = 2
_NS = 16
_NW = _NC * _NS
_GRP = 128
_NBUF = 4
_NACC = 8
_RU = 4


def _make_bias_kernel(n_groups_total):
    n_groups_w = n_groups_total // _NW
    mesh = plsc.VectorSubcoreMesh(core_axis_name="c", subcore_axis_name="s")

    @functools.partial(
        pl.kernel,
        mesh=mesh,
        out_type=jax.ShapeDtypeStruct((_NW, _LANES), jnp.float32),
        compiler_params=pltpu.CompilerParams(use_tc_tiling_on_sc=False),
        scratch_types=[
            pltpu.VMEM((n_groups_w, _GRP), jnp.int32),
            pltpu.VMEM((n_groups_w, _GRP), jnp.int32),
            pltpu.VMEM((_NBUF, _GRP), jnp.float32),
            pltpu.VMEM((_NBUF, _GRP), jnp.float32),
            pltpu.VMEM((_LANES,), jnp.float32),
        ] + [pltpu.SemaphoreType.DMA] * _NBUF,
    )
    def bias_kernel(tok_hbm, head_hbm, ub_hbm, vb_hbm,
                    out_hbm, idx_t, idx_h, ub, vb, acc_v, *sems):
        wid = lax.axis_index("s") * _NC + lax.axis_index("c")
        g_base = wid * n_groups_w
        pltpu.sync_copy(tok_hbm.at[pl.ds(g_base, n_groups_w)], idx_t)
        pltpu.sync_copy(head_hbm.at[pl.ds(g_base, n_groups_w)], idx_h)

        def issue(gi, b):
            pltpu.async_copy(ub_hbm.at[idx_t.at[gi]], ub.at[b], sems[b])
            pltpu.async_copy(vb_hbm.at[idx_h.at[gi]], vb.at[b], sems[b])

        def drain(b):
            pltpu.make_async_copy(ub_hbm.at[pl.ds(0, _GRP)],
                                  ub.at[b], sems[b]).wait()
            pltpu.make_async_copy(vb_hbm.at[pl.ds(0, _GRP)],
                                  vb.at[b], sems[b]).wait()

        def compute(b, accs):
            a = list(accs)
            for j in range(_GRP // _LANES):
                sl = pl.ds(j * _LANES, _LANES)
                a[j] = a[j] + ub[b, sl] + vb[b, sl]
            return tuple(a)

        for b in range(_NBUF - 1):
            issue(b, b)

        def quad_body(q, accs):
            for b in range(_NBUF):
                gi = q * _NBUF + b
                issue(jnp.minimum(gi + _NBUF - 1, n_groups_w - 1),
                      (b + _NBUF - 1) % _NBUF)
                drain(b)
                accs = compute(b, accs)
            return accs

        accs = lax.fori_loop(
            0, n_groups_w // _NBUF, quad_body,
            tuple(jnp.zeros((_LANES,), jnp.float32) for _ in range(_NACC)))
        for b in range(_NBUF - 1):
            drain(b)

        total = accs[0]
        for a in accs[1:]:
            total = total + a
        acc_v[...] = total
        pltpu.sync_copy(acc_v, out_hbm.at[wid])

    return bias_kernel


def _make_sc_kernel(n_groups_total):
    n_groups_w = n_groups_total // _NW
    assert n_groups_w % _NBUF == 0
    mesh = plsc.VectorSubcoreMesh(core_axis_name="c", subcore_axis_name="s")

    @functools.partial(
        pl.kernel,
        mesh=mesh,
        out_type=jax.ShapeDtypeStruct((_NW, _LANES), jnp.float32),
        compiler_params=pltpu.CompilerParams(use_tc_tiling_on_sc=False),
        scratch_types=[
            pltpu.VMEM((n_groups_w, _GRP), jnp.int32),
            pltpu.VMEM((n_groups_w, _GRP), jnp.int32),
            pltpu.VMEM((_NBUF, _GRP, _DIM), jnp.float32),
            pltpu.VMEM((_NBUF, _GRP, _DIM), jnp.float32),
            pltpu.VMEM((_LANES,), jnp.float32),
        ] + [pltpu.SemaphoreType.DMA] * _NBUF,
    )
    def sc_kernel(tok_hbm, head_hbm, u_hbm, v_hbm,
                  out_hbm, idx_t, idx_h, u_rows, v_rows, acc_v,
                  *sems):
        wid = lax.axis_index("s") * _NC + lax.axis_index("c")
        g_base = wid * n_groups_w
        pltpu.sync_copy(tok_hbm.at[pl.ds(g_base, n_groups_w)], idx_t)
        pltpu.sync_copy(head_hbm.at[pl.ds(g_base, n_groups_w)], idx_h)

        def issue(gi, b):
            pltpu.async_copy(u_hbm.at[idx_t.at[gi]], u_rows.at[b], sems[b])
            pltpu.async_copy(v_hbm.at[idx_h.at[gi]], v_rows.at[b], sems[b])

        def drain(b):
            pltpu.make_async_copy(u_hbm.at[pl.ds(0, _GRP)],
                                  u_rows.at[b], sems[b]).wait()
            pltpu.make_async_copy(v_hbm.at[pl.ds(0, _GRP)],
                                  v_rows.at[b], sems[b]).wait()

        def compute(b, accs):
            def row_body(i, a):
                a = list(a)
                for r in range(_RU):
                    for s in range(_DIM // _LANES):
                        sl = pl.ds(s * _LANES, _LANES)
                        k = (r % 2) * (_DIM // _LANES) + s
                        a[k] = a[k] + (u_rows[b, i * _RU + r, sl] *
                                       v_rows[b, i * _RU + r, sl])
                return tuple(a)

            return lax.fori_loop(0, _GRP // _RU, row_body, accs)

        for b in range(_NBUF - 1):
            issue(b, b)

        def quad_body(q, accs):
            for b in range(_NBUF):
                gi = q * _NBUF + b
                issue(jnp.minimum(gi + _NBUF - 1, n_groups_w - 1),
                      (b + _NBUF - 1) % _NBUF)
                drain(b)
                accs = compute(b, accs)
            return accs

        accs = lax.fori_loop(
            0, n_groups_w // _NBUF, quad_body,
            tuple(jnp.zeros((_LANES,), jnp.float32) for _ in range(_NACC)))
        for b in range(_NBUF - 1):
            drain(b)

        total = accs[0]
        for a in accs[1:]:
            total = total + a
        acc_v[...] = total
        pltpu.sync_copy(acc_v, out_hbm.at[wid])

    return sc_kernel


def kernel(tokens_batch, heads_batch, U, Ubias, V, Vbias):
    b, l = tokens_batch.shape
    n = b * l
    n_groups_total = n // _GRP
    tok = tokens_batch.reshape(n_groups_total, _GRP).astype(jnp.int32)
    head = heads_batch.reshape(n_groups_total, _GRP).astype(jnp.int32)
    ub_flat = Ubias.reshape(-1)
    vb_flat = Vbias.reshape(-1)
    bias_partials = _make_bias_kernel(n_groups_total)(
        tok, head, ub_flat, vb_flat)
    partials = _make_sc_kernel(n_groups_total)(tok, head, U, V)
    return jnp.sum(partials) + jnp.sum(bias_partials)
